# initial kernel scaffold (unmeasured)
import jax
import jax.numpy as jnp
from jax import lax
from jax.experimental import pallas as pl
from jax.experimental.pallas import tpu as pltpu


def kernel(
    x,
):
    def body(*refs):
        pass

    out_shape = jax.ShapeDtypeStruct(..., jnp.float32)
    return pl.pallas_call(body, out_shape=out_shape)(...)


# baseline (device time: 17784 ns/iter reference)
import jax
import jax.numpy as jnp
from jax import lax
from jax.experimental import pallas as pl
from jax.experimental.pallas import tpu as pltpu

B = 512


def kernel(x):
    def body(x_ref, out_ref, send_sem, recv_sem):
        my_x = lax.axis_index("x")
        my_y = lax.axis_index("y")
        peer_y = 1 - my_y

        barrier_sem = pltpu.get_barrier_semaphore()
        pl.semaphore_signal(
            barrier_sem,
            inc=1,
            device_id=(my_x, peer_y),
            device_id_type=pl.DeviceIdType.MESH,
        )
        pl.semaphore_wait(barrier_sem, 1)

        out_ref[pl.ds(my_y * B, B), :] = x_ref[:, pl.ds(my_y * B, B)]

        rdma = pltpu.make_async_remote_copy(
            src_ref=x_ref.at[:, pl.ds(peer_y * B, B)],
            dst_ref=out_ref.at[pl.ds(my_y * B, B), :],
            send_sem=send_sem,
            recv_sem=recv_sem,
            device_id=(my_x, peer_y),
            device_id_type=pl.DeviceIdType.MESH,
        )
        rdma.start()
        rdma.wait()

    return pl.pallas_call(
        body,
        out_shape=jax.ShapeDtypeStruct((2 * B, B), jnp.float32),
        in_specs=[pl.BlockSpec(memory_space=pltpu.VMEM)],
        out_specs=pl.BlockSpec(memory_space=pltpu.VMEM),
        scratch_shapes=[
            pltpu.SemaphoreType.DMA,
            pltpu.SemaphoreType.DMA,
        ],
        compiler_params=pltpu.CompilerParams(collective_id=0),
    )(x)


# device time: 16075 ns/iter; 1.1063x vs baseline; 1.1063x over previous
import jax
import jax.numpy as jnp
from jax import lax
from jax.experimental import pallas as pl
from jax.experimental.pallas import tpu as pltpu

B = 512
HALF = 256
K = 4
CH = HALF // K


def kernel(x):
    def body(x_ref, out_ref, send_a, recv_a, send_b, recv_b):
        my_x = lax.axis_index("x")
        my_y = lax.axis_index("y")
        peer_x = 1 - my_x
        peer_y = 1 - my_y

        barrier_sem = pltpu.get_barrier_semaphore()
        for dev in ((my_x, peer_y), (peer_x, my_y)):
            pl.semaphore_signal(
                barrier_sem,
                inc=1,
                device_id=dev,
                device_id_type=pl.DeviceIdType.MESH,
            )
        pl.semaphore_wait(barrier_sem, 2)

        rdma_a = []
        for c in range(K):
            src_row = my_x * HALF + c * CH
            dst_row = my_y * B + my_x * HALF + c * CH
            a = pltpu.make_async_remote_copy(
                src_ref=x_ref.at[pl.ds(src_row, CH), pl.ds(peer_y * B, B)],
                dst_ref=out_ref.at[pl.ds(dst_row, CH), :],
                send_sem=send_a.at[c],
                recv_sem=recv_a.at[c],
                device_id=(my_x, peer_y),
                device_id_type=pl.DeviceIdType.MESH,
            )
            a.start()
            rdma_a.append(a)

        out_ref[pl.ds(my_y * B, B), :] = x_ref[:, pl.ds(my_y * B, B)]

        rdma_b = []
        for c in range(K):
            rdma_a[c].wait_recv()
            row = peer_y * B + my_x * HALF + c * CH
            b = pltpu.make_async_remote_copy(
                src_ref=out_ref.at[pl.ds(row, CH), :],
                dst_ref=out_ref.at[pl.ds(row, CH), :],
                send_sem=send_b.at[c],
                recv_sem=recv_b.at[c],
                device_id=(peer_x, my_y),
                device_id_type=pl.DeviceIdType.MESH,
            )
            b.start()
            rdma_b.append(b)

        for c in range(K):
            rdma_b[c].wait_recv()
        for c in range(K):
            rdma_a[c].wait_send()
            rdma_b[c].wait_send()

    return pl.pallas_call(
        body,
        out_shape=jax.ShapeDtypeStruct((2 * B, B), jnp.float32),
        in_specs=[pl.BlockSpec(memory_space=pltpu.VMEM)],
        out_specs=pl.BlockSpec(memory_space=pltpu.VMEM),
        scratch_shapes=[
            pltpu.SemaphoreType.DMA((K,)),
            pltpu.SemaphoreType.DMA((K,)),
            pltpu.SemaphoreType.DMA((K,)),
            pltpu.SemaphoreType.DMA((K,)),
        ],
        compiler_params=pltpu.CompilerParams(collective_id=0),
    )(x)


# device time: 15751 ns/iter; 1.1291x vs baseline; 1.0206x over previous
import jax
import jax.numpy as jnp
from jax import lax
from jax.experimental import pallas as pl
from jax.experimental.pallas import tpu as pltpu

B = 512
HALF = 256
K = 8
CH = HALF // K


def kernel(x):
    def body(x_ref, out_ref, local_sem, send_a, recv_a, send_b, recv_b):
        my_x = lax.axis_index("x")
        my_y = lax.axis_index("y")
        peer_x = 1 - my_x
        peer_y = 1 - my_y

        local_copy = pltpu.make_async_copy(
            x_ref.at[:, pl.ds(my_y * B, B)],
            out_ref.at[pl.ds(my_y * B, B), :],
            local_sem,
        )
        local_copy.start()

        barrier_sem = pltpu.get_barrier_semaphore()
        for dev in ((my_x, peer_y), (peer_x, my_y)):
            pl.semaphore_signal(
                barrier_sem,
                inc=1,
                device_id=dev,
                device_id_type=pl.DeviceIdType.MESH,
            )
        pl.semaphore_wait(barrier_sem, 2)

        rdma_a = []
        for c in range(K):
            src_row = my_x * HALF + c * CH
            dst_row = my_y * B + my_x * HALF + c * CH
            a = pltpu.make_async_remote_copy(
                src_ref=x_ref.at[pl.ds(src_row, CH), pl.ds(peer_y * B, B)],
                dst_ref=out_ref.at[pl.ds(dst_row, CH), :],
                send_sem=send_a.at[c],
                recv_sem=recv_a.at[c],
                device_id=(my_x, peer_y),
                device_id_type=pl.DeviceIdType.MESH,
            )
            a.start()
            rdma_a.append(a)

        rdma_b = []
        for c in range(K):
            rdma_a[c].wait_recv()
            row = peer_y * B + my_x * HALF + c * CH
            b = pltpu.make_async_remote_copy(
                src_ref=out_ref.at[pl.ds(row, CH), :],
                dst_ref=out_ref.at[pl.ds(row, CH), :],
                send_sem=send_b.at[c],
                recv_sem=recv_b.at[c],
                device_id=(peer_x, my_y),
                device_id_type=pl.DeviceIdType.MESH,
            )
            b.start()
            rdma_b.append(b)

        for c in range(K):
            rdma_b[c].wait_recv()
        local_copy.wait()
        for c in range(K):
            rdma_a[c].wait_send()
            rdma_b[c].wait_send()

    return pl.pallas_call(
        body,
        out_shape=jax.ShapeDtypeStruct((2 * B, B), jnp.float32),
        in_specs=[pl.BlockSpec(memory_space=pltpu.VMEM)],
        out_specs=pl.BlockSpec(memory_space=pltpu.VMEM),
        scratch_shapes=[
            pltpu.SemaphoreType.DMA,
            pltpu.SemaphoreType.DMA((K,)),
            pltpu.SemaphoreType.DMA((K,)),
            pltpu.SemaphoreType.DMA((K,)),
            pltpu.SemaphoreType.DMA((K,)),
        ],
        compiler_params=pltpu.CompilerParams(collective_id=0),
    )(x)


# device time: 15507 ns/iter; 1.1468x vs baseline; 1.0157x over previous
import jax
import jax.numpy as jnp
from jax import lax
from jax.experimental import pallas as pl
from jax.experimental.pallas import tpu as pltpu

B = 512
HALF = 256
K = 16
CH = HALF // K


def kernel(x):
    def body(x_ref, out_ref, x_ready, local_sem, send_a, recv_a, send_b, recv_b):
        my_x = lax.axis_index("x")
        my_y = lax.axis_index("y")
        peer_x = 1 - my_x
        peer_y = 1 - my_y

        barrier_sem = pltpu.get_barrier_semaphore()
        pl.semaphore_signal(
            barrier_sem,
            inc=1,
            device_id=(my_x, peer_y),
            device_id_type=pl.DeviceIdType.MESH,
        )
        pl.semaphore_signal(
            x_ready,
            inc=1,
            device_id=(peer_x, my_y),
            device_id_type=pl.DeviceIdType.MESH,
        )

        local_copy = pltpu.make_async_copy(
            x_ref.at[:, pl.ds(my_y * B, B)],
            out_ref.at[pl.ds(my_y * B, B), :],
            local_sem,
        )
        local_copy.start()

        pl.semaphore_wait(barrier_sem, 1)

        rdma_a = []
        for c in range(K):
            src_row = my_x * HALF + c * CH
            dst_row = my_y * B + my_x * HALF + c * CH
            a = pltpu.make_async_remote_copy(
                src_ref=x_ref.at[pl.ds(src_row, CH), pl.ds(peer_y * B, B)],
                dst_ref=out_ref.at[pl.ds(dst_row, CH), :],
                send_sem=send_a.at[c],
                recv_sem=recv_a.at[c],
                device_id=(my_x, peer_y),
                device_id_type=pl.DeviceIdType.MESH,
            )
            a.start()
            rdma_a.append(a)

        rdma_b = []
        for c in range(K):
            rdma_a[c].wait_recv()
            if c == 0:
                pl.semaphore_wait(x_ready, 1)
            row = peer_y * B + my_x * HALF + c * CH
            b = pltpu.make_async_remote_copy(
                src_ref=out_ref.at[pl.ds(row, CH), :],
                dst_ref=out_ref.at[pl.ds(row, CH), :],
                send_sem=send_b.at[c],
                recv_sem=recv_b.at[c],
                device_id=(peer_x, my_y),
                device_id_type=pl.DeviceIdType.MESH,
            )
            b.start()
            rdma_b.append(b)

        for c in range(K):
            rdma_b[c].wait_recv()
        local_copy.wait()
        for c in range(K):
            rdma_a[c].wait_send()
            rdma_b[c].wait_send()

    return pl.pallas_call(
        body,
        out_shape=jax.ShapeDtypeStruct((2 * B, B), jnp.float32),
        in_specs=[pl.BlockSpec(memory_space=pltpu.VMEM)],
        out_specs=pl.BlockSpec(memory_space=pltpu.VMEM),
        scratch_shapes=[
            pltpu.SemaphoreType.REGULAR,
            pltpu.SemaphoreType.DMA,
            pltpu.SemaphoreType.DMA((K,)),
            pltpu.SemaphoreType.DMA((K,)),
            pltpu.SemaphoreType.DMA((K,)),
            pltpu.SemaphoreType.DMA((K,)),
        ],
        compiler_params=pltpu.CompilerParams(collective_id=0),
    )(x)
